# baseline (speedup 1.0000x reference)
import jax
import jax.numpy as jnp
from jax import lax
from jax.experimental import pallas as pl
from jax.experimental.pallas import tpu as pltpu

_D_IN, _D_HID, _D_OUT = 20, 15, 10
_TM = 32768
_NPH = _TM // 4096


def _mlp_l1_kernel(x_ref, t_ref, w1bd_ref, w2bd_ref, out_ref,
                   xpk_a, xpk_b, hpk_a, hpk_b, tpk_a, tpk_b):
    i = pl.program_id(1)

    @pl.when(i == 0)
    def _init():
        out_ref[...] = jnp.zeros_like(out_ref)
        r = lax.broadcasted_iota(jnp.int32, (192, 256), 0)
        ones_rows = jnp.where(r % 24 == _D_IN, 1.0, 0.0).astype(jnp.float32)
        xpk_a[...] = ones_rows
        xpk_b[...] = ones_rows
        tpk_a[...] = jnp.zeros_like(tpk_a)
        tpk_b[...] = jnp.zeros_like(tpk_b)

    w1bd = w1bd_ref[...]
    w2bd = w2bd_ref[...]
    acc = jnp.zeros((8, 256), jnp.float32)
    for ph in range(_NPH):
        hpk = hpk_a if ph % 2 == 0 else hpk_b
        tpk = tpk_a if ph % 2 == 0 else tpk_b
        for g in range(2):
            xpk = xpk_a if g == 0 else xpk_b
            base = ph * 16 + g * 8
            for c in range(8):
                xpk[pl.ds(24 * c, _D_IN), :] = (
                    x_ref[:, pl.ds((base + c) * 256, 256)])
            h = jnp.dot(w1bd, xpk[...], preferred_element_type=jnp.float32)
            hpk[pl.ds(128 * g, 128), :] = jnp.maximum(h, 0.0)
        for c in range(16):
            tpk[pl.ds(16 * c, _D_OUT), :] = (
                t_ref[:, pl.ds((ph * 16 + c) * 256, 256)])
        y = jnp.dot(w2bd, hpk[...], preferred_element_type=jnp.float32)
        d = jnp.abs(y - tpk[...])
        acc = acc + jnp.sum(d.reshape(32, 8, 256), axis=0)
    out_ref[...] += (acc[:, :128] + acc[:, 128:])[None]


def kernel(xT, tT, w1t, b1, w2t, b2):
    B = xT.shape[1]
    nc = 2
    b_pad = nc * _TM * pl.cdiv(B, nc * _TM)
    w1f = w1t.astype(jnp.float32)
    b1f = b1.astype(jnp.float32)
    w2f = w2t.astype(jnp.float32)
    b2f = b2.astype(jnp.float32)
    if b_pad != B:
        y0 = w2f @ jnp.maximum(b1f, 0.0) + b2f
        xT = jnp.pad(xT, ((0, 0), (0, b_pad - B)))
        tT = jnp.concatenate(
            [tT, jnp.broadcast_to(y0[:, None], (_D_OUT, b_pad - B))], axis=1)
    ntpc = b_pad // (nc * _TM)

    blk1 = (jnp.zeros((16, 24), jnp.float32)
            .at[:_D_HID, :_D_IN].set(w1f)
            .at[:_D_HID, _D_IN].set(b1f)
            .at[_D_HID, _D_IN].set(1.0))
    w1bd = (jnp.eye(8, dtype=jnp.float32)[:, None, :, None]
            * blk1[None, :, None, :]).reshape(128, 192)
    blk2 = (jnp.zeros((16, 16), jnp.float32)
            .at[:_D_OUT, :_D_HID].set(w2f)
            .at[:_D_OUT, _D_HID].set(b2f))
    w2bd = (jnp.eye(16, dtype=jnp.float32)[:, None, :, None]
            * blk2[None, :, None, :]).reshape(256, 256)

    out = pl.pallas_call(
        _mlp_l1_kernel,
        out_shape=jax.ShapeDtypeStruct((nc, 8, 128), jnp.float32),
        grid=(nc, ntpc),
        in_specs=[
            pl.BlockSpec((_D_IN, _TM),
                         lambda c, i, ntpc=ntpc: (0, c * ntpc + i)),
            pl.BlockSpec((_D_OUT, _TM),
                         lambda c, i, ntpc=ntpc: (0, c * ntpc + i)),
            pl.BlockSpec((128, 192), lambda c, i: (0, 0)),
            pl.BlockSpec((256, 256), lambda c, i: (0, 0)),
        ],
        out_specs=pl.BlockSpec((1, 8, 128), lambda c, i: (c, 0, 0)),
        scratch_shapes=[
            pltpu.VMEM((192, 256), jnp.float32),
            pltpu.VMEM((192, 256), jnp.float32),
            pltpu.VMEM((256, 256), jnp.float32),
            pltpu.VMEM((256, 256), jnp.float32),
            pltpu.VMEM((256, 256), jnp.float32),
            pltpu.VMEM((256, 256), jnp.float32),
        ],
        compiler_params=pltpu.CompilerParams(
            dimension_semantics=("parallel", "arbitrary"),
        ),
        cost_estimate=pl.CostEstimate(
            flops=2 * b_pad * (_D_IN * _D_HID + _D_HID * _D_OUT),
            transcendentals=0,
            bytes_accessed=4 * b_pad * (_D_IN + _D_OUT)),
    )(xT, tT, w1bd, w2bd)

    return jnp.sum(out) * (1.0 / float(B * _D_OUT))


# speedup vs baseline: 1.0037x; 1.0037x over previous
import jax
import jax.numpy as jnp
from jax import lax
from jax.experimental import pallas as pl
from jax.experimental.pallas import tpu as pltpu

_D_IN, _D_HID, _D_OUT = 20, 15, 10
_TM = 32768
_W = 1024
_NPH = _TM // (16 * _W)


def _mlp_l1_kernel(x_ref, t_ref, w1bd_ref, w2bd_ref, out_ref,
                   xpk, tpk):
    i = pl.program_id(1)

    @pl.when(i == 0)
    def _init():
        out_ref[...] = jnp.zeros_like(out_ref)
        r = lax.broadcasted_iota(jnp.int32, (4, 192, _W), 1)
        xpk[...] = jnp.where(r % 24 == _D_IN, 1.0, 0.0).astype(jnp.float32)
        tpk[...] = jnp.zeros_like(tpk)

    w1bd = w1bd_ref[...]
    w2bd = w2bd_ref[...]
    acc = jnp.zeros((8, _W), jnp.float32)
    for ph in range(_NPH):
        hs = []
        for g in range(2):
            xp = xpk.at[(2 * ph + g) % 4]
            base = ph * 16 + g * 8
            for c in range(8):
                xp[pl.ds(24 * c, _D_IN), :] = (
                    x_ref[:, pl.ds((base + c) * _W, _W)])
            h = jnp.dot(w1bd, xp[...], preferred_element_type=jnp.float32)
            hs.append(jnp.maximum(h, 0.0))
        tp = tpk.at[ph % 2]
        for c in range(16):
            tp[pl.ds(16 * c, _D_OUT), :] = (
                t_ref[:, pl.ds((ph * 16 + c) * _W, _W)])
        y = jnp.dot(w2bd, jnp.concatenate(hs, axis=0),
                    preferred_element_type=jnp.float32)
        d = jnp.abs(y - tp[...])
        acc = acc + jnp.sum(d.reshape(32, 8, _W), axis=0)
    out_ref[...] += acc[None]


def kernel(xT, tT, w1t, b1, w2t, b2):
    B = xT.shape[1]
    nc = 2
    b_pad = nc * _TM * pl.cdiv(B, nc * _TM)
    w1f = w1t.astype(jnp.float32)
    b1f = b1.astype(jnp.float32)
    w2f = w2t.astype(jnp.float32)
    b2f = b2.astype(jnp.float32)
    if b_pad != B:
        y0 = w2f @ jnp.maximum(b1f, 0.0) + b2f
        xT = jnp.pad(xT, ((0, 0), (0, b_pad - B)))
        tT = jnp.concatenate(
            [tT, jnp.broadcast_to(y0[:, None], (_D_OUT, b_pad - B))], axis=1)
    ntpc = b_pad // (nc * _TM)

    blk1 = (jnp.zeros((16, 24), jnp.float32)
            .at[:_D_HID, :_D_IN].set(w1f)
            .at[:_D_HID, _D_IN].set(b1f)
            .at[_D_HID, _D_IN].set(1.0))
    w1bd = (jnp.eye(8, dtype=jnp.float32)[:, None, :, None]
            * blk1[None, :, None, :]).reshape(128, 192)
    blk2 = (jnp.zeros((16, 16), jnp.float32)
            .at[:_D_OUT, :_D_HID].set(w2f)
            .at[:_D_OUT, _D_HID].set(b2f))
    w2bd = (jnp.eye(16, dtype=jnp.float32)[:, None, :, None]
            * blk2[None, :, None, :]).reshape(256, 256)

    out = pl.pallas_call(
        _mlp_l1_kernel,
        out_shape=jax.ShapeDtypeStruct((nc, 8, _W), jnp.float32),
        grid=(nc, ntpc),
        in_specs=[
            pl.BlockSpec((_D_IN, _TM),
                         lambda c, i, ntpc=ntpc: (0, c * ntpc + i)),
            pl.BlockSpec((_D_OUT, _TM),
                         lambda c, i, ntpc=ntpc: (0, c * ntpc + i)),
            pl.BlockSpec((128, 192), lambda c, i: (0, 0)),
            pl.BlockSpec((256, 256), lambda c, i: (0, 0)),
        ],
        out_specs=pl.BlockSpec((1, 8, _W), lambda c, i: (c, 0, 0)),
        scratch_shapes=[
            pltpu.VMEM((4, 192, _W), jnp.float32),
            pltpu.VMEM((2, 256, _W), jnp.float32),
        ],
        compiler_params=pltpu.CompilerParams(
            dimension_semantics=("parallel", "arbitrary"),
        ),
        cost_estimate=pl.CostEstimate(
            flops=2 * b_pad * (_D_IN * _D_HID + _D_HID * _D_OUT),
            transcendentals=0,
            bytes_accessed=4 * b_pad * (_D_IN + _D_OUT)),
    )(xT, tT, w1bd, w2bd)

    return jnp.sum(out) * (1.0 / float(B * _D_OUT))


# speedup vs baseline: 1.4169x; 1.4116x over previous
import jax
import jax.numpy as jnp
from jax import lax
from jax.experimental import pallas as pl
from jax.experimental.pallas import tpu as pltpu

_D_IN, _D_HID, _D_OUT = 20, 15, 10
_TM = 32768
_W = 1024
_NPH = _TM // (16 * _W)


def _mlp_l1_kernel(x_ref, t_ref, blk1_ref, blk2_ref, out_ref,
                   xpk, tpk, w1bd_s, w2bd_s):
    i = pl.program_id(1)

    @pl.when(i == 0)
    def _init():
        out_ref[...] = jnp.zeros_like(out_ref)
        r = lax.broadcasted_iota(jnp.int32, (4, 192, _W), 1)
        xpk[...] = jnp.where(r % 24 == _D_IN, 1.0, 0.0).astype(jnp.float32)
        tpk[...] = jnp.zeros_like(tpk)
        base1 = jnp.pad(blk1_ref[...], ((0, 112), (0, 168)))
        w1bd_s[...] = base1 + sum(
            jnp.roll(base1, (16 * c, 24 * c), axis=(0, 1))
            for c in range(1, 8))
        base2 = jnp.pad(blk2_ref[...], ((0, 240), (0, 240)))
        w2bd_s[...] = base2 + sum(
            jnp.roll(base2, (16 * c, 16 * c), axis=(0, 1))
            for c in range(1, 16))

    w1bd = w1bd_s[...]
    w2bd = w2bd_s[...]
    acc = jnp.zeros((8, _W), jnp.float32)
    for ph in range(_NPH):
        hs = []
        for g in range(2):
            xp = xpk.at[(2 * ph + g) % 4]
            base = ph * 16 + g * 8
            for c in range(8):
                xp[pl.ds(24 * c, _D_IN), :] = (
                    x_ref[:, pl.ds((base + c) * _W, _W)])
            h = jnp.dot(w1bd, xp[...], preferred_element_type=jnp.float32)
            hs.append(jnp.maximum(h, 0.0))
        tp = tpk.at[ph % 2]
        for c in range(16):
            tp[pl.ds(16 * c, _D_OUT), :] = (
                t_ref[:, pl.ds((ph * 16 + c) * _W, _W)])
        y = jnp.dot(w2bd, jnp.concatenate(hs, axis=0),
                    preferred_element_type=jnp.float32)
        d = jnp.abs(y - tp[...])
        acc = acc + jnp.sum(d.reshape(32, 8, _W), axis=0)
    out_ref[...] += acc[None]


def kernel(xT, tT, w1t, b1, w2t, b2):
    B = xT.shape[1]
    nc = 2
    b_pad = nc * _TM * pl.cdiv(B, nc * _TM)
    w1f = w1t.astype(jnp.float32)
    b1f = b1.astype(jnp.float32)
    w2f = w2t.astype(jnp.float32)
    b2f = b2.astype(jnp.float32)
    if b_pad != B:
        y0 = w2f @ jnp.maximum(b1f, 0.0) + b2f
        xT = jnp.pad(xT, ((0, 0), (0, b_pad - B)))
        tT = jnp.concatenate(
            [tT, jnp.broadcast_to(y0[:, None], (_D_OUT, b_pad - B))], axis=1)
    ntpc = b_pad // (nc * _TM)

    import numpy as _np
    corner = _np.zeros((16, 24), _np.float32)
    corner[_D_HID, _D_IN] = 1.0
    blk1 = (jnp.pad(w1f, ((0, 1), (0, 4)))
            + jnp.pad(b1f[:, None], ((0, 1), (_D_IN, 3)))
            + corner)
    blk2 = (jnp.pad(w2f, ((0, 6), (0, 1)))
            + jnp.pad(b2f[:, None], ((0, 6), (_D_HID, 0))))

    out = pl.pallas_call(
        _mlp_l1_kernel,
        out_shape=jax.ShapeDtypeStruct((nc, 8, _W), jnp.float32),
        grid=(nc, ntpc),
        in_specs=[
            pl.BlockSpec((_D_IN, _TM),
                         lambda c, i, ntpc=ntpc: (0, c * ntpc + i)),
            pl.BlockSpec((_D_OUT, _TM),
                         lambda c, i, ntpc=ntpc: (0, c * ntpc + i)),
            pl.BlockSpec((16, 24), lambda c, i: (0, 0)),
            pl.BlockSpec((16, 16), lambda c, i: (0, 0)),
        ],
        out_specs=pl.BlockSpec((1, 8, _W), lambda c, i: (c, 0, 0)),
        scratch_shapes=[
            pltpu.VMEM((4, 192, _W), jnp.float32),
            pltpu.VMEM((2, 256, _W), jnp.float32),
            pltpu.VMEM((128, 192), jnp.float32),
            pltpu.VMEM((256, 256), jnp.float32),
        ],
        compiler_params=pltpu.CompilerParams(
            dimension_semantics=("parallel", "arbitrary"),
        ),
        cost_estimate=pl.CostEstimate(
            flops=2 * b_pad * (_D_IN * _D_HID + _D_HID * _D_OUT),
            transcendentals=0,
            bytes_accessed=4 * b_pad * (_D_IN + _D_OUT)),
    )(xT, tT, blk1, blk2)

    return jnp.sum(out) * (1.0 / float(B * _D_OUT))


# speedup vs baseline: 1.4642x; 1.0334x over previous
import jax
import jax.numpy as jnp
from jax import lax
from jax.experimental import pallas as pl
from jax.experimental.pallas import tpu as pltpu

_D_IN, _D_HID, _D_OUT = 20, 15, 10
_TM = 65536
_W = 1024
_NPH = _TM // (16 * _W)


def _mlp_l1_kernel(x_ref, t_ref, blk1_ref, blk2_ref, out_ref,
                   xpk, tpk, w1bd_s, w2bd_s):
    i = pl.program_id(1)

    @pl.when(i == 0)
    def _init():
        out_ref[...] = jnp.zeros_like(out_ref)
        r = lax.broadcasted_iota(jnp.int32, (4, 192, _W), 1)
        xpk[...] = jnp.where(r % 24 == _D_IN, 1.0, 0.0).astype(jnp.float32)
        tpk[...] = jnp.zeros_like(tpk)
        base1 = jnp.pad(blk1_ref[...], ((0, 112), (0, 168)))
        w1bd_s[...] = base1 + sum(
            jnp.roll(base1, (16 * c, 24 * c), axis=(0, 1))
            for c in range(1, 8))
        base2 = jnp.pad(blk2_ref[...], ((0, 240), (0, 240)))
        w2bd_s[...] = base2 + sum(
            jnp.roll(base2, (16 * c, 16 * c), axis=(0, 1))
            for c in range(1, 16))

    w1bd = w1bd_s[...]
    w2bd = w2bd_s[...]
    acc = jnp.zeros((8, _W), jnp.float32)
    for ph in range(_NPH):
        hs = []
        for g in range(2):
            xp = xpk.at[(2 * ph + g) % 4]
            base = ph * 16 + g * 8
            for c in range(8):
                xp[pl.ds(24 * c, _D_IN), :] = (
                    x_ref[:, pl.ds((base + c) * _W, _W)])
            h = jnp.dot(w1bd, xp[...], preferred_element_type=jnp.float32)
            hs.append(jnp.maximum(h, 0.0))
        tp = tpk.at[ph % 2]
        for c in range(16):
            tp[pl.ds(16 * c, _D_OUT), :] = (
                t_ref[:, pl.ds((ph * 16 + c) * _W, _W)])
        y = jnp.dot(w2bd, jnp.concatenate(hs, axis=0),
                    preferred_element_type=jnp.float32)
        d = jnp.abs(y - tp[...])
        acc = acc + jnp.sum(d.reshape(32, 8, _W), axis=0)
    out_ref[...] += acc[None]


def kernel(xT, tT, w1t, b1, w2t, b2):
    B = xT.shape[1]
    nc = 2
    b_pad = nc * _TM * pl.cdiv(B, nc * _TM)
    w1f = w1t.astype(jnp.float32)
    b1f = b1.astype(jnp.float32)
    w2f = w2t.astype(jnp.float32)
    b2f = b2.astype(jnp.float32)
    if b_pad != B:
        y0 = w2f @ jnp.maximum(b1f, 0.0) + b2f
        xT = jnp.pad(xT, ((0, 0), (0, b_pad - B)))
        tT = jnp.concatenate(
            [tT, jnp.broadcast_to(y0[:, None], (_D_OUT, b_pad - B))], axis=1)
    ntpc = b_pad // (nc * _TM)

    import numpy as _np
    corner = _np.zeros((16, 24), _np.float32)
    corner[_D_HID, _D_IN] = 1.0
    blk1 = (jnp.pad(w1f, ((0, 1), (0, 4)))
            + jnp.pad(b1f[:, None], ((0, 1), (_D_IN, 3)))
            + corner)
    blk2 = (jnp.pad(w2f, ((0, 6), (0, 1)))
            + jnp.pad(b2f[:, None], ((0, 6), (_D_HID, 0))))

    out = pl.pallas_call(
        _mlp_l1_kernel,
        out_shape=jax.ShapeDtypeStruct((nc, 8, _W), jnp.float32),
        grid=(nc, ntpc),
        in_specs=[
            pl.BlockSpec((_D_IN, _TM),
                         lambda c, i, ntpc=ntpc: (0, c * ntpc + i)),
            pl.BlockSpec((_D_OUT, _TM),
                         lambda c, i, ntpc=ntpc: (0, c * ntpc + i)),
            pl.BlockSpec((16, 24), lambda c, i: (0, 0)),
            pl.BlockSpec((16, 16), lambda c, i: (0, 0)),
        ],
        out_specs=pl.BlockSpec((1, 8, _W), lambda c, i: (c, 0, 0)),
        scratch_shapes=[
            pltpu.VMEM((4, 192, _W), jnp.float32),
            pltpu.VMEM((2, 256, _W), jnp.float32),
            pltpu.VMEM((128, 192), jnp.float32),
            pltpu.VMEM((256, 256), jnp.float32),
        ],
        compiler_params=pltpu.CompilerParams(
            dimension_semantics=("parallel", "arbitrary"),
        ),
        cost_estimate=pl.CostEstimate(
            flops=2 * b_pad * (_D_IN * _D_HID + _D_HID * _D_OUT),
            transcendentals=0,
            bytes_accessed=4 * b_pad * (_D_IN + _D_OUT)),
    )(xT, tT, blk1, blk2)

    return jnp.sum(out) * (1.0 / float(B * _D_OUT))
